# baseline (device time: 13541 ns/iter reference)
import jax
import jax.numpy as jnp
from jax import lax
from jax.experimental import pallas as pl
from jax.experimental.pallas import tpu as pltpu


def kernel(x):
    m, n = x.shape
    half = m // 2

    def body(x_ref, out_ref, sbuf, zrecv, xrecv, sems):
        my_x = lax.axis_index("x")
        my_y = lax.axis_index("y")
        my_z = lax.axis_index("z")
        z_peer = (my_x, my_y, 1 - my_z)
        x_peer = (1 - my_x, my_y, my_z)

        sbuf[...] = x_ref[pl.ds(0, half), :].astype(jnp.bfloat16)

        barrier_sem = pltpu.get_barrier_semaphore()
        for nbr in (z_peer, x_peer):
            pl.semaphore_signal(
                barrier_sem, inc=1, device_id=nbr,
                device_id_type=pl.DeviceIdType.MESH,
            )
        pl.semaphore_wait(barrier_sem, 2)

        rz = pltpu.make_async_remote_copy(
            src_ref=sbuf, dst_ref=zrecv,
            send_sem=sems.at[0], recv_sem=sems.at[1],
            device_id=z_peer, device_id_type=pl.DeviceIdType.MESH,
        )
        rx = pltpu.make_async_remote_copy(
            src_ref=sbuf, dst_ref=xrecv,
            send_sem=sems.at[2], recv_sem=sems.at[3],
            device_id=x_peer, device_id_type=pl.DeviceIdType.MESH,
        )
        rz.start()
        rx.start()
        rz.wait()
        rx.wait()

        out_ref[pl.ds(0, half), :] = zrecv[...] + sbuf[...]
        out_ref[pl.ds(half, half), :] = xrecv[...] + sbuf[...]

    return pl.pallas_call(
        body,
        out_shape=jax.ShapeDtypeStruct((m, n), jnp.bfloat16),
        in_specs=[pl.BlockSpec(memory_space=pltpu.VMEM)],
        out_specs=pl.BlockSpec(memory_space=pltpu.VMEM),
        scratch_shapes=[
            pltpu.VMEM((half, n), jnp.bfloat16),
            pltpu.VMEM((half, n), jnp.bfloat16),
            pltpu.VMEM((half, n), jnp.bfloat16),
            pltpu.SemaphoreType.DMA((4,)),
        ],
        compiler_params=pltpu.CompilerParams(collective_id=0),
    )(x)
